# baseline (speedup 1.0000x reference)
import functools

import jax
import jax.numpy as jnp
from jax import lax
from jax.experimental import pallas as pl
from jax.experimental.pallas import tpu as pltpu
from jax.experimental.pallas import tpu_sc as plsc

TOPK = 2
NEG = -1e30


def _router_body(caps, x_ref, wg_ref, out_ref, wrep_ref, cnt_ref):
    TB, D = x_ref.shape
    E, CAP, TRASH = caps
    i = pl.program_id(0)

    @pl.when(i == 0)
    def _():
        cnt_ref[...] = jnp.zeros_like(cnt_ref)

    xb = x_ref[...]
    logits = jax.lax.dot_general(
        xb, wg_ref[...], (((1,), (0,)), ((), ())),
        preferred_element_type=jnp.float32)

    lane = jax.lax.broadcasted_iota(jnp.int32, (TB, E), 1)
    m1 = jnp.max(logits, axis=1, keepdims=True)
    e0 = jnp.min(jnp.where(logits == m1, lane, E + 1), axis=1, keepdims=True)
    masked = jnp.where(lane == e0, NEG, logits)
    m2 = jnp.max(masked, axis=1, keepdims=True)
    e1 = jnp.min(jnp.where(masked == m2, lane, E + 1), axis=1, keepdims=True)

    t = jnp.exp(m2 - m1)
    w0 = 1.0 / (1.0 + t)
    w1 = 1.0 - w0

    A = (lane == e0).astype(jnp.float32)
    B = (lane == e1).astype(jnp.float32)
    r = jax.lax.broadcasted_iota(jnp.int32, (TB, TB), 0)
    c = jax.lax.broadcasted_iota(jnp.int32, (TB, TB), 1)
    S = (c < r).astype(jnp.float32)
    AB = A + B
    pre = jax.lax.dot_general(S, AB, (((1,), (0,)), ((), ())),
                              preferred_element_type=jnp.float32)
    carry = cnt_ref[0:1, :]
    base = pre + carry
    pos0 = jnp.sum(base * A, axis=1, keepdims=True)
    pos1 = jnp.sum((base + A) * B, axis=1, keepdims=True)
    cnt_ref[0:1, :] = carry + jnp.sum(AB, axis=0, keepdims=True)

    e0f = e0.astype(jnp.float32)
    e1f = e1.astype(jnp.float32)
    f0 = e0f * CAP + pos0
    f1 = e1f * CAP + pos1
    v0 = pos0 < CAP
    v1 = pos1 < CAP
    s0 = jnp.where(v0, f0, TRASH)
    s1 = jnp.where(v1, f1, TRASH)
    g0 = jnp.where(v0, f0, 0.0)
    g1 = jnp.where(v1, f1, 0.0)
    w0 = jnp.where(v0, w0, 0.0)
    w1 = jnp.where(v1, w1, 0.0)

    ocol = jax.lax.broadcasted_iota(jnp.int32, (TB, 8), 1)
    out = (jnp.where(ocol == 0, s0, 0.0) + jnp.where(ocol == 1, s1, 0.0)
           + jnp.where(ocol == 2, g0, 0.0) + jnp.where(ocol == 3, g1, 0.0)
           + jnp.where(ocol == 4, w0, 0.0) + jnp.where(ocol == 5, w1, 0.0))
    out_ref[...] = out
    wcol = jax.lax.broadcasted_iota(jnp.int32, (TB, 128), 1)
    wrep_ref[...] = jnp.where(wcol < 16, w0, jnp.where(wcol < 32, w1, 0.0))


def _router(x, Wg, E, CAP, TRASH):
    T, D = x.shape
    TB = 512
    grid = (T // TB,)
    return pl.pallas_call(
        functools.partial(_router_body, (E, CAP, TRASH)),
        grid=grid,
        in_specs=[
            pl.BlockSpec((TB, D), lambda i: (i, 0)),
            pl.BlockSpec((D, E), lambda i: (0, 0)),
        ],
        out_specs=[pl.BlockSpec((TB, 8), lambda i: (i, 0)),
                   pl.BlockSpec((TB, 128), lambda i: (i, 0))],
        out_shape=[jax.ShapeDtypeStruct((T, 8), jnp.float32),
                   jax.ShapeDtypeStruct((T, 128), jnp.float32)],
        scratch_shapes=[pltpu.VMEM((8, E), jnp.float32)],
        compiler_params=pltpu.CompilerParams(
            dimension_semantics=("arbitrary",)),
    )(x, Wg)


def _make_dispatch(T, D, NROWS):
    info = plsc.get_sparse_core_info()
    NC, NS = info.num_cores, info.num_subcores
    NW = NC * NS
    tok_per_w = T // NW
    SUB = 32
    NSUB = tok_per_w // SUB
    mesh = plsc.VectorSubcoreMesh(core_axis_name="c", subcore_axis_name="s")

    @functools.partial(
        pl.kernel, mesh=mesh,
        out_type=jax.ShapeDtypeStruct((NROWS, D), jnp.float32),
        scratch_types=[
            pltpu.VMEM((SUB, D), jnp.float32),
            pltpu.VMEM((SUB, D), jnp.float32),
            pltpu.VMEM((SUB,), jnp.int32),
            pltpu.VMEM((SUB,), jnp.int32),
            pltpu.VMEM((SUB,), jnp.int32),
            pltpu.VMEM((SUB,), jnp.int32),
            pltpu.SemaphoreType.DMA,
            pltpu.SemaphoreType.DMA,
            pltpu.SemaphoreType.DMA,
            pltpu.SemaphoreType.DMA,
        ],
    )
    def dispatch(x_hbm, ridx_hbm, buf_hbm, xva, xvb, i0a, i1a, i0b, i1b,
                 semxa, semxb, semsa, semsb):
        wid = lax.axis_index("s") * NC + lax.axis_index("c")
        base = wid * tok_per_w
        slots = ((xva, i0a, i1a, semxa, semsa),
                 (xvb, i0b, i1b, semxb, semsb))

        def fire_load(slot, tb):
            xv, i0v, i1v, semx, _ = slot
            pltpu.sync_copy(ridx_hbm.at[0, pl.ds(tb, SUB)], i0v)
            pltpu.sync_copy(ridx_hbm.at[1, pl.ds(tb, SUB)], i1v)
            cpx = pltpu.make_async_copy(x_hbm.at[pl.ds(tb, SUB)], xv, semx)
            cpx.start()
            return cpx

        def fire_scatter(slot):
            xv, i0v, i1v, _, sems = slot
            cp0 = pltpu.make_async_copy(xv, buf_hbm.at[i0v], sems)
            cp0.start()
            cp1 = pltpu.make_async_copy(xv, buf_hbm.at[i1v], sems)
            cp1.start()
            return cp0, cp1

        loadh = [fire_load(slots[0], base), None]
        scath = [None, None]
        for sub in range(NSUB):
            cur = sub & 1
            nxt = 1 - cur
            if scath[nxt] is not None:
                scath[nxt][0].wait()
                scath[nxt][1].wait()
                scath[nxt] = None
            if sub + 1 < NSUB:
                loadh[nxt] = fire_load(slots[nxt], base + (sub + 1) * SUB)
            loadh[cur].wait()
            scath[cur] = fire_scatter(slots[cur])
        for h in scath:
            if h is not None:
                h[0].wait()
                h[1].wait()

    return dispatch


def _ffn_body(fb, x_ref, w1_ref, w3_ref, w2_ref, y_ref):
    f = pl.program_id(1)
    xb = x_ref[...].astype(jnp.bfloat16)
    h = jax.lax.dot_general(xb, w1_ref[0].astype(jnp.bfloat16),
                            (((1,), (1,)), ((), ())),
                            preferred_element_type=jnp.float32)
    u = jax.lax.dot_general(xb, w3_ref[0].astype(jnp.bfloat16),
                            (((1,), (1,)), ((), ())),
                            preferred_element_type=jnp.float32)
    act = (h * (1.0 / (1.0 + jnp.exp(-h))) * u).astype(jnp.bfloat16)
    w2b = w2_ref[0, :, pl.ds(f * fb, fb)].astype(jnp.bfloat16)
    yp = jax.lax.dot_general(act, w2b, (((1,), (1,)), ((), ())),
                             preferred_element_type=jnp.float32)

    @pl.when(f == 0)
    def _():
        y_ref[...] = yp

    @pl.when(f > 0)
    def _():
        y_ref[...] = y_ref[...] + yp


def _ffn(buf, W1, W3, W2, E, CAP):
    _, DFF, D = W1.shape
    NF = 2
    FB = DFF // NF
    return pl.pallas_call(
        functools.partial(_ffn_body, FB),
        grid=(E, NF),
        in_specs=[
            pl.BlockSpec((CAP, D), lambda e, f: (e, 0)),
            pl.BlockSpec((1, FB, D), lambda e, f: (e, f, 0)),
            pl.BlockSpec((1, FB, D), lambda e, f: (e, f, 0)),
            pl.BlockSpec((1, D, DFF), lambda e, f: (e, 0, 0)),
        ],
        out_specs=pl.BlockSpec((CAP, D), lambda e, f: (e, 0)),
        out_shape=jax.ShapeDtypeStruct((E * CAP, D), jnp.float32),
        compiler_params=pltpu.CompilerParams(
            dimension_semantics=("arbitrary", "arbitrary")),
    )(buf, W1, W3, W2)


def _make_combine(T, D, NROWS):
    info = plsc.get_sparse_core_info()
    NC, NS = info.num_cores, info.num_subcores
    NW = NC * NS
    tok_per_w = T // NW
    SUB = 16
    NSUB = tok_per_w // SUB
    NL = 16
    mesh = plsc.VectorSubcoreMesh(core_axis_name="c", subcore_axis_name="s")

    @functools.partial(
        pl.kernel, mesh=mesh,
        out_type=jax.ShapeDtypeStruct((T, D), jnp.float32),
        scratch_types=[
            pltpu.VMEM((SUB, D), jnp.float32),
            pltpu.VMEM((SUB, D), jnp.float32),
            pltpu.VMEM((SUB, D), jnp.float32),
            pltpu.VMEM((SUB, D), jnp.float32),
            pltpu.VMEM((SUB, D), jnp.float32),
            pltpu.VMEM((SUB,), jnp.int32),
            pltpu.VMEM((SUB,), jnp.int32),
            pltpu.VMEM((SUB,), jnp.int32),
            pltpu.VMEM((SUB,), jnp.int32),
            pltpu.VMEM((SUB, 128), jnp.float32),
            pltpu.VMEM((SUB, 128), jnp.float32),
            pltpu.SemaphoreType.DMA,
            pltpu.SemaphoreType.DMA,
        ],
    )
    def combine(y_hbm, ridx_hbm, wrep_hbm, out_hbm,
                y0a, y1a, y0b, y1b, accv,
                g0a, g1a, g0b, g1b, wva, wvb, sema, semb):
        wid = lax.axis_index("s") * NC + lax.axis_index("c")
        base = wid * tok_per_w
        slots = ((y0a, y1a, g0a, g1a, wva, sema),
                 (y0b, y1b, g0b, g1b, wvb, semb))

        def fire(slot, tb):
            y0v, y1v, g0v, g1v, wv, sem = slot
            pltpu.sync_copy(ridx_hbm.at[2, pl.ds(tb, SUB)], g0v)
            pltpu.sync_copy(ridx_hbm.at[3, pl.ds(tb, SUB)], g1v)
            pltpu.sync_copy(wrep_hbm.at[pl.ds(tb, SUB)], wv)
            cp0 = pltpu.make_async_copy(y_hbm.at[g0v], y0v, sem)
            cp0.start()
            cp1 = pltpu.make_async_copy(y_hbm.at[g1v], y1v, sem)
            cp1.start()
            return cp0, cp1

        z = jnp.zeros((NL,), jnp.float32)
        handles = [fire(slots[0], base), None]
        for sub in range(NSUB):
            cur = sub & 1
            if sub + 1 < NSUB:
                handles[1 - cur] = fire(slots[1 - cur], base + (sub + 1) * SUB)
            cp0, cp1 = handles[cur]
            cp0.wait()
            cp1.wait()
            y0v, y1v, _, _, wv, _ = slots[cur]

            def row_body(r, _, y0v=y0v, y1v=y1v, wv=wv):
                w0b = wv[r, pl.ds(0, NL)]
                w1b = wv[r, pl.ds(NL, NL)]
                for cidx in range(D // NL):
                    sl = pl.ds(cidx * NL, NL)
                    a = jnp.where(w0b != 0.0, y0v[r, sl] * w0b, z)
                    b = jnp.where(w1b != 0.0, y1v[r, sl] * w1b, z)
                    accv[r, sl] = a + b
                return 0

            lax.fori_loop(0, SUB, row_body, 0)
            pltpu.sync_copy(accv, out_hbm.at[pl.ds(base + sub * SUB, SUB)])

    return combine


def kernel(hidden_states, Wg, W1, W3, W2):
    T, D = hidden_states.shape
    E = Wg.shape[1]
    CAP = 2 * (T * TOPK // E)
    TRASH = E * CAP
    NROWS = E * CAP + 8

    rout, wrep = _router(hidden_states, Wg, E, CAP, TRASH)
    ridx = rout[:, :4].astype(jnp.int32).T

    dispatch = _make_dispatch(T, D, NROWS)
    buf = dispatch(hidden_states, ridx)

    y = _ffn(buf, W1, W3, W2, E, CAP)

    combine = _make_combine(T, D, NROWS)
    return combine(y, ridx, wrep)


# speedup vs baseline: 1.1597x; 1.1597x over previous
import functools

import jax
import jax.numpy as jnp
from jax import lax
from jax.experimental import pallas as pl
from jax.experimental.pallas import tpu as pltpu
from jax.experimental.pallas import tpu_sc as plsc

TOPK = 2
NEG = -1e30


def _router_body(caps, x_ref, wg_ref, out_ref, wrep_ref, cnt_ref):
    TB, D = x_ref.shape
    E, CAP, TRASH = caps
    i = pl.program_id(0)

    @pl.when(i == 0)
    def _():
        cnt_ref[...] = jnp.zeros_like(cnt_ref)

    xb = x_ref[...]
    logits = jax.lax.dot_general(
        xb, wg_ref[...], (((1,), (0,)), ((), ())),
        preferred_element_type=jnp.float32)

    lane = jax.lax.broadcasted_iota(jnp.int32, (TB, E), 1)
    m1 = jnp.max(logits, axis=1, keepdims=True)
    e0 = jnp.min(jnp.where(logits == m1, lane, E + 1), axis=1, keepdims=True)
    masked = jnp.where(lane == e0, NEG, logits)
    m2 = jnp.max(masked, axis=1, keepdims=True)
    e1 = jnp.min(jnp.where(masked == m2, lane, E + 1), axis=1, keepdims=True)

    t = jnp.exp(m2 - m1)
    w0 = 1.0 / (1.0 + t)
    w1 = 1.0 - w0

    A = (lane == e0).astype(jnp.float32)
    B = (lane == e1).astype(jnp.float32)
    r = jax.lax.broadcasted_iota(jnp.int32, (TB, TB), 0)
    c = jax.lax.broadcasted_iota(jnp.int32, (TB, TB), 1)
    S = (c < r).astype(jnp.float32)
    AB = A + B
    pre = jax.lax.dot_general(S, AB, (((1,), (0,)), ((), ())),
                              preferred_element_type=jnp.float32)
    carry = cnt_ref[0:1, :]
    base = pre + carry
    pos0 = jnp.sum(base * A, axis=1, keepdims=True)
    pos1 = jnp.sum((base + A) * B, axis=1, keepdims=True)
    cnt_ref[0:1, :] = carry + jnp.sum(AB, axis=0, keepdims=True)

    e0f = e0.astype(jnp.float32)
    e1f = e1.astype(jnp.float32)
    f0 = e0f * CAP + pos0
    f1 = e1f * CAP + pos1
    v0 = pos0 < CAP
    v1 = pos1 < CAP
    s0 = jnp.where(v0, f0, TRASH)
    s1 = jnp.where(v1, f1, TRASH)
    g0 = jnp.where(v0, f0, 0.0)
    g1 = jnp.where(v1, f1, 0.0)
    w0 = jnp.where(v0, w0, 0.0)
    w1 = jnp.where(v1, w1, 0.0)

    ocol = jax.lax.broadcasted_iota(jnp.int32, (TB, 8), 1)
    out = (jnp.where(ocol == 0, s0, 0.0) + jnp.where(ocol == 1, s1, 0.0)
           + jnp.where(ocol == 2, g0, 0.0) + jnp.where(ocol == 3, g1, 0.0)
           + jnp.where(ocol == 4, w0, 0.0) + jnp.where(ocol == 5, w1, 0.0))
    out_ref[...] = out
    wcol = jax.lax.broadcasted_iota(jnp.int32, (TB, 128), 1)
    wrep_ref[...] = jnp.where(wcol < 16, w0, jnp.where(wcol < 32, w1, 0.0))


def _router(x, Wg, E, CAP, TRASH):
    T, D = x.shape
    TB = 512
    grid = (T // TB,)
    return pl.pallas_call(
        functools.partial(_router_body, (E, CAP, TRASH)),
        grid=grid,
        in_specs=[
            pl.BlockSpec((TB, D), lambda i: (i, 0)),
            pl.BlockSpec((D, E), lambda i: (0, 0)),
        ],
        out_specs=[pl.BlockSpec((TB, 8), lambda i: (i, 0)),
                   pl.BlockSpec((TB, 128), lambda i: (i, 0))],
        out_shape=[jax.ShapeDtypeStruct((T, 8), jnp.float32),
                   jax.ShapeDtypeStruct((T, 128), jnp.float32)],
        scratch_shapes=[pltpu.VMEM((8, E), jnp.float32)],
        compiler_params=pltpu.CompilerParams(
            dimension_semantics=("arbitrary",)),
    )(x, Wg)


def _make_dispatch(T, D, NROWS):
    info = plsc.get_sparse_core_info()
    NC, NS = info.num_cores, info.num_subcores
    NW = NC * NS
    tok_per_w = T // NW
    SUB = 32
    NSUB = tok_per_w // SUB
    mesh = plsc.VectorSubcoreMesh(core_axis_name="c", subcore_axis_name="s")

    @functools.partial(
        pl.kernel, mesh=mesh,
        out_type=jax.ShapeDtypeStruct((NROWS, D), jnp.float32),
        scratch_types=[
            pltpu.VMEM((SUB, D), jnp.float32),
            pltpu.VMEM((SUB, D), jnp.float32),
            pltpu.VMEM((SUB,), jnp.int32),
            pltpu.VMEM((SUB,), jnp.int32),
            pltpu.VMEM((SUB,), jnp.int32),
            pltpu.VMEM((SUB,), jnp.int32),
            pltpu.SemaphoreType.DMA,
            pltpu.SemaphoreType.DMA,
            pltpu.SemaphoreType.DMA,
            pltpu.SemaphoreType.DMA,
        ],
    )
    def dispatch(x_hbm, ridx_hbm, buf_hbm, xva, xvb, i0a, i1a, i0b, i1b,
                 semxa, semxb, semsa, semsb):
        wid = lax.axis_index("s") * NC + lax.axis_index("c")
        base = wid * tok_per_w
        slots = ((xva, i0a, i1a, semxa, semsa),
                 (xvb, i0b, i1b, semxb, semsb))

        def fire_load(slot, tb):
            xv, i0v, i1v, semx, _ = slot
            pltpu.sync_copy(ridx_hbm.at[0, pl.ds(tb, SUB)], i0v)
            pltpu.sync_copy(ridx_hbm.at[1, pl.ds(tb, SUB)], i1v)
            cpx = pltpu.make_async_copy(x_hbm.at[pl.ds(tb, SUB)], xv, semx)
            cpx.start()
            return cpx

        def fire_scatter(slot):
            xv, i0v, i1v, _, sems = slot
            cp0 = pltpu.make_async_copy(xv, buf_hbm.at[i0v], sems)
            cp0.start()
            cp1 = pltpu.make_async_copy(xv, buf_hbm.at[i1v], sems)
            cp1.start()
            return cp0, cp1

        loadh = [fire_load(slots[0], base), None]
        scath = [None, None]
        for sub in range(NSUB):
            cur = sub & 1
            nxt = 1 - cur
            if scath[nxt] is not None:
                scath[nxt][0].wait()
                scath[nxt][1].wait()
                scath[nxt] = None
            if sub + 1 < NSUB:
                loadh[nxt] = fire_load(slots[nxt], base + (sub + 1) * SUB)
            loadh[cur].wait()
            scath[cur] = fire_scatter(slots[cur])
        for h in scath:
            if h is not None:
                h[0].wait()
                h[1].wait()

    return dispatch


def _ffn_body(x_ref, w1_ref, w3_ref, w2_ref, y_ref):
    xb = x_ref[...].astype(jnp.bfloat16)
    h = jax.lax.dot_general(xb, w1_ref[0].astype(jnp.bfloat16),
                            (((1,), (1,)), ((), ())),
                            preferred_element_type=jnp.float32)
    u = jax.lax.dot_general(xb, w3_ref[0].astype(jnp.bfloat16),
                            (((1,), (1,)), ((), ())),
                            preferred_element_type=jnp.float32)
    act = (h * (1.0 / (1.0 + jnp.exp(-h))) * u).astype(jnp.bfloat16)
    y_ref[...] = jax.lax.dot_general(
        act, w2_ref[0].astype(jnp.bfloat16), (((1,), (1,)), ((), ())),
        preferred_element_type=jnp.float32)


def _ffn(buf, W1, W3, W2, E, CAP):
    _, DFF, D = W1.shape
    return pl.pallas_call(
        _ffn_body,
        grid=(E,),
        in_specs=[
            pl.BlockSpec((CAP, D), lambda e: (e, 0)),
            pl.BlockSpec((1, DFF, D), lambda e: (e, 0, 0)),
            pl.BlockSpec((1, DFF, D), lambda e: (e, 0, 0)),
            pl.BlockSpec((1, D, DFF), lambda e: (e, 0, 0)),
        ],
        out_specs=pl.BlockSpec((CAP, D), lambda e: (e, 0)),
        out_shape=jax.ShapeDtypeStruct((E * CAP, D), jnp.float32),
        compiler_params=pltpu.CompilerParams(
            dimension_semantics=("arbitrary",)),
    )(buf, W1, W3, W2)


def _make_combine(T, D, NROWS):
    info = plsc.get_sparse_core_info()
    NC, NS = info.num_cores, info.num_subcores
    NW = NC * NS
    tok_per_w = T // NW
    SUB = 16
    NSUB = tok_per_w // SUB
    NL = 16
    mesh = plsc.VectorSubcoreMesh(core_axis_name="c", subcore_axis_name="s")

    @functools.partial(
        pl.kernel, mesh=mesh,
        out_type=jax.ShapeDtypeStruct((T, D), jnp.float32),
        scratch_types=[
            pltpu.VMEM((SUB, D), jnp.float32),
            pltpu.VMEM((SUB, D), jnp.float32),
            pltpu.VMEM((SUB, D), jnp.float32),
            pltpu.VMEM((SUB, D), jnp.float32),
            pltpu.VMEM((SUB, D), jnp.float32),
            pltpu.VMEM((SUB,), jnp.int32),
            pltpu.VMEM((SUB,), jnp.int32),
            pltpu.VMEM((SUB,), jnp.int32),
            pltpu.VMEM((SUB,), jnp.int32),
            pltpu.VMEM((SUB, 128), jnp.float32),
            pltpu.VMEM((SUB, 128), jnp.float32),
            pltpu.SemaphoreType.DMA,
            pltpu.SemaphoreType.DMA,
        ],
    )
    def combine(y_hbm, ridx_hbm, wrep_hbm, out_hbm,
                y0a, y1a, y0b, y1b, accv,
                g0a, g1a, g0b, g1b, wva, wvb, sema, semb):
        wid = lax.axis_index("s") * NC + lax.axis_index("c")
        base = wid * tok_per_w
        slots = ((y0a, y1a, g0a, g1a, wva, sema),
                 (y0b, y1b, g0b, g1b, wvb, semb))

        def fire(slot, tb):
            y0v, y1v, g0v, g1v, wv, sem = slot
            pltpu.sync_copy(ridx_hbm.at[2, pl.ds(tb, SUB)], g0v)
            pltpu.sync_copy(ridx_hbm.at[3, pl.ds(tb, SUB)], g1v)
            pltpu.sync_copy(wrep_hbm.at[pl.ds(tb, SUB)], wv)
            cp0 = pltpu.make_async_copy(y_hbm.at[g0v], y0v, sem)
            cp0.start()
            cp1 = pltpu.make_async_copy(y_hbm.at[g1v], y1v, sem)
            cp1.start()
            return cp0, cp1

        z = jnp.zeros((NL,), jnp.float32)
        handles = [fire(slots[0], base), None]
        for sub in range(NSUB):
            cur = sub & 1
            if sub + 1 < NSUB:
                handles[1 - cur] = fire(slots[1 - cur], base + (sub + 1) * SUB)
            cp0, cp1 = handles[cur]
            cp0.wait()
            cp1.wait()
            y0v, y1v, _, _, wv, _ = slots[cur]

            def row_body(r, _, y0v=y0v, y1v=y1v, wv=wv):
                w0b = wv[r, pl.ds(0, NL)]
                w1b = wv[r, pl.ds(NL, NL)]
                for cidx in range(D // NL):
                    sl = pl.ds(cidx * NL, NL)
                    a = jnp.where(w0b != 0.0, y0v[r, sl] * w0b, z)
                    b = jnp.where(w1b != 0.0, y1v[r, sl] * w1b, z)
                    accv[r, sl] = a + b
                return 0

            lax.fori_loop(0, SUB, row_body, 0)
            pltpu.sync_copy(accv, out_hbm.at[pl.ds(base + sub * SUB, SUB)])

    return combine


def kernel(hidden_states, Wg, W1, W3, W2):
    T, D = hidden_states.shape
    E = Wg.shape[1]
    CAP = 2 * (T * TOPK // E)
    TRASH = E * CAP
    NROWS = E * CAP + 8

    rout, wrep = _router(hidden_states, Wg, E, CAP, TRASH)
    ridx = rout[:, :4].astype(jnp.int32).T

    dispatch = _make_dispatch(T, D, NROWS)
    buf = dispatch(hidden_states, ridx)

    y = _ffn(buf, W1, W3, W2, E, CAP)

    combine = _make_combine(T, D, NROWS)
    return combine(y, ridx, wrep)
